# Initial kernel scaffold; baseline (speedup 1.0000x reference)
import jax
import jax.numpy as jnp
from jax import lax
from jax.experimental import pallas as pl
from jax.experimental.pallas import tpu as pltpu
from jax.experimental.pallas import tpu_sc as plsc

NC = 2
NS = 16
L = 16
NW = NC * NS

DIM = 128
NCHUNK = 4
C = 128


def _sc_body(pf_hbm, ps_hbm, emb_hbm, w_hbm, b_hbm, out_hbm,
             idx_f, idx_s, fbufs, sbufs, wv, bv, ob, sems):
    wid = lax.axis_index("s") * NC + lax.axis_index("c")
    rpw = NCHUNK * C
    base = wid * rpw

    pltpu.sync_copy(pf_hbm.at[pl.ds(wid * NCHUNK, NCHUNK)], idx_f)
    pltpu.sync_copy(ps_hbm.at[pl.ds(wid * NCHUNK, NCHUNK)], idx_s)
    pltpu.sync_copy(w_hbm, wv)
    pltpu.sync_copy(b_hbm, bv)

    def start(c):
        par = c % 2
        hf = pltpu.async_copy(emb_hbm.at[idx_f.at[c]], fbufs[par], sems[2 * par])
        hs = pltpu.async_copy(emb_hbm.at[idx_s.at[c]], sbufs[par], sems[2 * par + 1])
        return hf, hs

    w_regs = [wv[pl.ds(k * L, L)] for k in range(8)]
    b_reg = bv[...]
    lanes = lax.iota(jnp.int32, L)

    def compute_chunk(c):
        fb = fbufs[c % 2]
        sb = sbufs[c % 2]

        def group_body(g, _):
            def row_body(r, gvec):
                row = g * L + r
                acc = jnp.zeros((L,), jnp.float32)
                for k in range(8):
                    fv = fb[row, pl.ds(k * L, L)]
                    sv = sb[row, pl.ds(k * L, L)]
                    acc = acc + fv * sv * w_regs[k]
                dot = jnp.sum(acc)
                return jnp.where(lanes == r, dot, gvec)

            gvec = lax.fori_loop(0, L, row_body, jnp.zeros((L,), jnp.float32))
            x = gvec + b_reg
            y = 1.0 / (1.0 + jnp.exp(-x))
            ob[pl.ds(c * C + g * L, L)] = y
            return 0

        lax.fori_loop(0, C // L, group_body, 0)

    handles = [None] * NCHUNK
    handles[0] = start(0)
    for c in range(NCHUNK):
        if c + 1 < NCHUNK:
            handles[c + 1] = start(c + 1)
        handles[c][0].wait()
        handles[c][1].wait()
        compute_chunk(c)

    pltpu.sync_copy(ob, out_hbm.at[pl.ds(base, rpw)])


@jax.jit
def _word2vec_sc(pf2, ps2, emb, w, bvec):
    B = pf2.shape[0] * pf2.shape[1]
    mesh = plsc.VectorSubcoreMesh(core_axis_name="c", subcore_axis_name="s")
    kern = pl.kernel(
        _sc_body,
        out_type=jax.ShapeDtypeStruct((B,), jnp.float32),
        mesh=mesh,
        scratch_types=dict(
            idx_f=pltpu.VMEM((NCHUNK, C), jnp.int32),
            idx_s=pltpu.VMEM((NCHUNK, C), jnp.int32),
            fbufs=[pltpu.VMEM((C, DIM), jnp.float32) for _ in range(2)],
            sbufs=[pltpu.VMEM((C, DIM), jnp.float32) for _ in range(2)],
            wv=pltpu.VMEM((DIM,), jnp.float32),
            bv=pltpu.VMEM((L,), jnp.float32),
            ob=pltpu.VMEM((NCHUNK * C,), jnp.float32),
            sems=[pltpu.SemaphoreType.DMA for _ in range(4)],
        ),
    )
    return kern(pf2, ps2, emb, w, bvec)


def kernel(pair_first, pair_second, emb, W, b):
    B = pair_first.shape[0]
    pf2 = pair_first.reshape(NW * NCHUNK, C).astype(jnp.int32)
    ps2 = pair_second.reshape(NW * NCHUNK, C).astype(jnp.int32)
    w = W.reshape(DIM).astype(jnp.float32)
    bvec = jnp.broadcast_to(b.astype(jnp.float32).reshape(()), (L,))
    out = _word2vec_sc(pf2, ps2, emb, w, bvec)
    return out.reshape(B, 1)


# speedup vs baseline: 1.4532x; 1.4532x over previous
import jax
import jax.numpy as jnp
from jax import lax
from jax.experimental import pallas as pl
from jax.experimental.pallas import tpu as pltpu
from jax.experimental.pallas import tpu_sc as plsc

NC = 2
NS = 16
L = 16
NW = NC * NS

DIM = 128
NCHUNK = 4
C = 128


def _sc_body(pf_hbm, ps_hbm, emb_hbm, w_hbm, b_hbm, out_hbm,
             idx_f, idx_s, fbufs, sbufs, wv, bv, ob, sems):
    wid = lax.axis_index("s") * NC + lax.axis_index("c")
    rpw = NCHUNK * C
    base = wid * rpw

    pltpu.sync_copy(pf_hbm.at[pl.ds(wid * NCHUNK, NCHUNK)], idx_f)
    pltpu.sync_copy(ps_hbm.at[pl.ds(wid * NCHUNK, NCHUNK)], idx_s)
    pltpu.sync_copy(w_hbm, wv)
    pltpu.sync_copy(b_hbm, bv)

    def start(c):
        par = c % 2
        hf = pltpu.async_copy(emb_hbm.at[idx_f.at[c]], fbufs[par], sems[2 * par])
        hs = pltpu.async_copy(emb_hbm.at[idx_s.at[c]], sbufs[par], sems[2 * par + 1])
        return hf, hs

    w_regs = [wv[pl.ds(k * L, L)] for k in range(8)]
    b_reg = bv[...]
    lanes = lax.iota(jnp.int32, L)
    perms = [lanes ^ sh for sh in (8, 4, 2, 1)]

    gdn = lax.GatherDimensionNumbers(
        offset_dims=(), collapsed_slice_dims=(0,), start_index_map=(0,))

    def shuffle(v, p):
        return lax.gather(v, p[:, None], gdn, slice_sizes=(1,),
                          mode=lax.GatherScatterMode.PROMISE_IN_BOUNDS)

    def hsum(v):
        for p in perms:
            v = v + shuffle(v, p)
        return v

    def compute_chunk(c):
        fb = fbufs[c % 2]
        sb = sbufs[c % 2]

        def group_body(g, _):
            def row_body(r, gvec):
                row = g * L + r
                acc = jnp.zeros((L,), jnp.float32)
                for k in range(8):
                    fv = fb[row, pl.ds(k * L, L)]
                    sv = sb[row, pl.ds(k * L, L)]
                    acc = acc + fv * sv * w_regs[k]
                dot = hsum(acc)
                return jnp.where(lanes == r, dot, gvec)

            gvec = lax.fori_loop(0, L, row_body, jnp.zeros((L,), jnp.float32))
            x = gvec + b_reg
            y = 1.0 / (1.0 + jnp.exp(-x))
            ob[pl.ds(c * C + g * L, L)] = y
            return 0

        lax.fori_loop(0, C // L, group_body, 0)

    handles = [None] * NCHUNK
    handles[0] = start(0)
    for c in range(NCHUNK):
        if c + 1 < NCHUNK:
            handles[c + 1] = start(c + 1)
        handles[c][0].wait()
        handles[c][1].wait()
        compute_chunk(c)

    pltpu.sync_copy(ob, out_hbm.at[pl.ds(base, rpw)])


@jax.jit
def _word2vec_sc(pf2, ps2, emb, w, bvec):
    B = pf2.shape[0] * pf2.shape[1]
    mesh = plsc.VectorSubcoreMesh(core_axis_name="c", subcore_axis_name="s")
    kern = pl.kernel(
        _sc_body,
        out_type=jax.ShapeDtypeStruct((B,), jnp.float32),
        mesh=mesh,
        scratch_types=dict(
            idx_f=pltpu.VMEM((NCHUNK, C), jnp.int32),
            idx_s=pltpu.VMEM((NCHUNK, C), jnp.int32),
            fbufs=[pltpu.VMEM((C, DIM), jnp.float32) for _ in range(2)],
            sbufs=[pltpu.VMEM((C, DIM), jnp.float32) for _ in range(2)],
            wv=pltpu.VMEM((DIM,), jnp.float32),
            bv=pltpu.VMEM((L,), jnp.float32),
            ob=pltpu.VMEM((NCHUNK * C,), jnp.float32),
            sems=[pltpu.SemaphoreType.DMA for _ in range(4)],
        ),
    )
    return kern(pf2, ps2, emb, w, bvec)


def kernel(pair_first, pair_second, emb, W, b):
    B = pair_first.shape[0]
    pf2 = pair_first.reshape(NW * NCHUNK, C).astype(jnp.int32)
    ps2 = pair_second.reshape(NW * NCHUNK, C).astype(jnp.int32)
    w = W.reshape(DIM).astype(jnp.float32)
    bvec = jnp.broadcast_to(b.astype(jnp.float32).reshape(()), (L,))
    out = _word2vec_sc(pf2, ps2, emb, w, bvec)
    return out.reshape(B, 1)


# speedup vs baseline: 1.5190x; 1.0453x over previous
import jax
import jax.numpy as jnp
from jax import lax
from jax.experimental import pallas as pl
from jax.experimental.pallas import tpu as pltpu
from jax.experimental.pallas import tpu_sc as plsc

NC = 2
NS = 16
L = 16
NW = NC * NS

DIM = 128
NCHUNK = 8
C = 64
NBUF = 4


def _sc_body(pf_hbm, ps_hbm, emb_hbm, w_hbm, b_hbm, out_hbm,
             idx_f, idx_s, fbufs, sbufs, wv, bv, ob, sems, psems):
    wid = lax.axis_index("s") * NC + lax.axis_index("c")
    rpw = NCHUNK * C
    base = wid * rpw

    h_if = pltpu.async_copy(pf_hbm.at[pl.ds(wid * NCHUNK, NCHUNK)], idx_f, psems[0])
    h_is = pltpu.async_copy(ps_hbm.at[pl.ds(wid * NCHUNK, NCHUNK)], idx_s, psems[1])
    h_w = pltpu.async_copy(w_hbm, wv, psems[2])
    h_b = pltpu.async_copy(b_hbm, bv, psems[3])
    h_if.wait()
    h_is.wait()

    def start(c):
        par = c % NBUF
        hf = pltpu.async_copy(emb_hbm.at[idx_f.at[c]], fbufs[par], sems[2 * par])
        hs = pltpu.async_copy(emb_hbm.at[idx_s.at[c]], sbufs[par], sems[2 * par + 1])
        return hf, hs

    handles = [None] * NCHUNK
    for c in range(NBUF):
        handles[c] = start(c)

    h_w.wait()
    h_b.wait()

    w_regs = [wv[pl.ds(k * L, L)] for k in range(8)]
    b_reg = bv[...]
    lanes = lax.iota(jnp.int32, L)
    perms = [lanes ^ sh for sh in (8, 4, 2, 1)]

    gdn = lax.GatherDimensionNumbers(
        offset_dims=(), collapsed_slice_dims=(0,), start_index_map=(0,))

    def shuffle(v, p):
        return lax.gather(v, p[:, None], gdn, slice_sizes=(1,),
                          mode=lax.GatherScatterMode.PROMISE_IN_BOUNDS)

    def hsum(v):
        for p in perms:
            v = v + shuffle(v, p)
        return v

    def compute_chunk(c):
        fb = fbufs[c % NBUF]
        sb = sbufs[c % NBUF]

        def group_body(g, _):
            def row_body(r, gvec):
                row = g * L + r
                acc = jnp.zeros((L,), jnp.float32)
                for k in range(8):
                    fv = fb[row, pl.ds(k * L, L)]
                    sv = sb[row, pl.ds(k * L, L)]
                    acc = acc + fv * sv * w_regs[k]
                dot = hsum(acc)
                return jnp.where(lanes == r, dot, gvec)

            gvec = lax.fori_loop(0, L, row_body, jnp.zeros((L,), jnp.float32))
            x = gvec + b_reg
            y = 1.0 / (1.0 + jnp.exp(-x))
            ob[pl.ds(c * C + g * L, L)] = y
            return 0

        lax.fori_loop(0, C // L, group_body, 0)

    for c in range(NCHUNK):
        handles[c][0].wait()
        handles[c][1].wait()
        compute_chunk(c)
        if c + NBUF < NCHUNK:
            handles[c + NBUF] = start(c + NBUF)

    pltpu.sync_copy(ob, out_hbm.at[pl.ds(base, rpw)])


@jax.jit
def _word2vec_sc(pf2, ps2, emb, w, bvec):
    B = pf2.shape[0] * pf2.shape[1]
    mesh = plsc.VectorSubcoreMesh(core_axis_name="c", subcore_axis_name="s")
    kern = pl.kernel(
        _sc_body,
        out_type=jax.ShapeDtypeStruct((B,), jnp.float32),
        mesh=mesh,
        scratch_types=dict(
            idx_f=pltpu.VMEM((NCHUNK, C), jnp.int32),
            idx_s=pltpu.VMEM((NCHUNK, C), jnp.int32),
            fbufs=[pltpu.VMEM((C, DIM), jnp.float32) for _ in range(NBUF)],
            sbufs=[pltpu.VMEM((C, DIM), jnp.float32) for _ in range(NBUF)],
            wv=pltpu.VMEM((DIM,), jnp.float32),
            bv=pltpu.VMEM((L,), jnp.float32),
            ob=pltpu.VMEM((NCHUNK * C,), jnp.float32),
            sems=[pltpu.SemaphoreType.DMA for _ in range(2 * NBUF)],
            psems=[pltpu.SemaphoreType.DMA for _ in range(4)],
        ),
    )
    return kern(pf2, ps2, emb, w, bvec)


def kernel(pair_first, pair_second, emb, W, b):
    B = pair_first.shape[0]
    pf2 = pair_first.reshape(NW * NCHUNK, C).astype(jnp.int32)
    ps2 = pair_second.reshape(NW * NCHUNK, C).astype(jnp.int32)
    w = W.reshape(DIM).astype(jnp.float32)
    bvec = jnp.broadcast_to(b.astype(jnp.float32).reshape(()), (L,))
    out = _word2vec_sc(pf2, ps2, emb, w, bvec)
    return out.reshape(B, 1)
